# baseline (speedup 1.0000x reference)
import functools

import jax
import jax.numpy as jnp
from jax import lax
from jax.experimental import pallas as pl
from jax.experimental.pallas import tpu as pltpu
from jax.experimental.pallas import tpu_sc as plsc

B, T = 16384, 200
D = 64
DP = 128
TP = 256
L = B * T
NC, NS = 2, 16
NW = NC * NS
ROWS_W = B // NW
NBUF = 2
VL = 16
RU = 4


def _make_kernel():
    mesh = plsc.VectorSubcoreMesh(core_axis_name="c", subcore_axis_name="s")

    @functools.partial(
        pl.kernel,
        mesh=mesh,
        out_type=jax.ShapeDtypeStruct((B, T, D), jnp.float32),
        scratch_types=[
            pltpu.VMEM((NBUF, 1, TP), jnp.int32),
            pltpu.VMEM((NBUF, TP, DP), jnp.float32),
            pltpu.VMEM((NBUF, T, D), jnp.float32),
            pltpu.SemaphoreType.DMA((NBUF,)),
            pltpu.SemaphoreType.DMA((NBUF,)),
            pltpu.SemaphoreType.DMA((NBUF,)),
        ],
    )
    def k(idx_hbm, table_hbm, out_hbm, idx_v, rows_a, rows_b,
          sem_i, sem_g, sem_o):
        wid = lax.axis_index("s") * NC + lax.axis_index("c")
        bw0 = wid * ROWS_W

        def idx_copy(g, b):
            return pltpu.make_async_copy(
                idx_hbm.at[bw0 + lax.min(g, ROWS_W - 1)],
                idx_v.at[b], sem_i.at[b])

        def gathers(b):
            return [
                pltpu.make_async_copy(
                    table_hbm.at[idx_v.at[b, 0, pl.ds(j * 128, 128)]],
                    rows_a.at[b, pl.ds(j * 128, 128)], sem_g.at[b])
                for j in range(TP // 128)
            ]

        def out_copy(bw, b):
            return pltpu.make_async_copy(
                rows_b.at[b], out_hbm.at[bw], sem_o.at[b])

        def repack(b):
            def rows(r, carry):
                for u in range(RU):
                    for v in range(D // VL):
                        s = pl.ds(v * VL, VL)
                        rows_b.at[b][r * RU + u, s] = rows_a.at[b][r * RU + u, s]
                return carry
            lax.fori_loop(0, T // RU, rows, 0)

        for b in range(NBUF):
            idx_copy(b, b).start()
        idx_copy(0, 0).wait()
        for cp in gathers(0):
            cp.start()

        def chunk_step(t, g, b):
            for cp in gathers(b):
                cp.wait()
            idx_copy(g + 2, b).start()
            idx_copy(g + 1, 1 - b).wait()
            for cp in gathers(1 - b):
                cp.start()
            @pl.when(t >= 1)
            def _():
                out_copy(bw0, b).wait()
            repack(b)
            out_copy(bw0 + g, b).start()

        def body(t, carry):
            for par in range(2):
                chunk_step(t, t * 2 + par, par)
            return carry

        lax.fori_loop(0, ROWS_W // 2, body, 0)

        for cp in gathers(0):
            cp.wait()
        for b in range(NBUF):
            out_copy(bw0, b).wait()
        idx_copy(ROWS_W - 1, 1).wait()

    return k


_gather_kernel = _make_kernel()


def kernel(indices, table):
    idx_p = jnp.pad(indices.astype(jnp.int32), ((0, 0), (0, TP - T)))
    idx_3d = idx_p.reshape(B, 1, TP)
    table_p = jnp.pad(table, ((0, 0), (0, DP - D)))
    return _gather_kernel(idx_3d, table_p)


# speedup vs baseline: 1.0014x; 1.0014x over previous
import functools

import jax
import jax.numpy as jnp
from jax import lax
from jax.experimental import pallas as pl
from jax.experimental.pallas import tpu as pltpu
from jax.experimental.pallas import tpu_sc as plsc

B, T = 16384, 200
D = 64
DP = 128
TP = 256
L = B * T
NC, NS = 2, 16
NW = NC * NS
ROWS_W = B // NW
NBUF = 2
VL = 16
RU = 4


def _make_kernel():
    mesh = plsc.VectorSubcoreMesh(core_axis_name="c", subcore_axis_name="s")

    @functools.partial(
        pl.kernel,
        mesh=mesh,
        out_type=jax.ShapeDtypeStruct((B, T, D), jnp.float32),
        scratch_types=[
            pltpu.VMEM((NBUF, TP // 128, 1, 128), jnp.int32),
            pltpu.VMEM((NBUF, TP, DP), jnp.float32),
            pltpu.VMEM((NBUF, T, D), jnp.float32),
            pltpu.SemaphoreType.DMA((NBUF,)),
            pltpu.SemaphoreType.DMA((NBUF,)),
            pltpu.SemaphoreType.DMA((NBUF,)),
        ],
    )
    def k(idx_hbm, table_hbm, out_hbm, idx_v, rows_a, rows_b,
          sem_i, sem_g, sem_o):
        wid = lax.axis_index("s") * NC + lax.axis_index("c")
        bw0 = wid * ROWS_W

        def idx_copy(g, b):
            return pltpu.make_async_copy(
                idx_hbm.at[bw0 + lax.min(g, ROWS_W - 1)],
                idx_v.at[b], sem_i.at[b])

        def gathers(b):
            return [
                pltpu.make_async_copy(
                    table_hbm.at[idx_v.at[b, j, 0]],
                    rows_a.at[b, pl.ds(j * 128, 128)], sem_g.at[b])
                for j in range(TP // 128)
            ]

        def out_copy(bw, b):
            return pltpu.make_async_copy(
                rows_b.at[b], out_hbm.at[bw], sem_o.at[b])

        def repack(b):
            def rows(r, carry):
                for u in range(RU):
                    for v in range(D // VL):
                        s = pl.ds(v * VL, VL)
                        rows_b.at[b][r * RU + u, s] = rows_a.at[b][r * RU + u, s]
                return carry
            lax.fori_loop(0, T // RU, rows, 0)

        for b in range(NBUF):
            idx_copy(b, b).start()
        idx_copy(0, 0).wait()
        for cp in gathers(0):
            cp.start()

        def chunk_step(t, g, b):
            for cp in gathers(b):
                cp.wait()
            idx_copy(g + 2, b).start()
            idx_copy(g + 1, 1 - b).wait()
            for cp in gathers(1 - b):
                cp.start()
            @pl.when(t >= 1)
            def _():
                out_copy(bw0, b).wait()
            repack(b)
            out_copy(bw0 + g, b).start()

        def body(t, carry):
            for par in range(2):
                chunk_step(t, t * 2 + par, par)
            return carry

        lax.fori_loop(0, ROWS_W // 2, body, 0)

        for cp in gathers(0):
            cp.wait()
        for b in range(NBUF):
            out_copy(bw0, b).wait()
        idx_copy(ROWS_W - 1, 1).wait()

    return k


_gather_kernel = _make_kernel()


def kernel(indices, table):
    idx_p = jnp.pad(indices.astype(jnp.int32), ((0, 0), (0, TP - T)))
    idx_3d = idx_p.reshape(B, TP // 128, 1, 128)
    table_p = jnp.pad(table, ((0, 0), (0, DP - D)))
    return _gather_kernel(idx_3d, table_p)
